# baseline (speedup 1.0000x reference)
import functools

import jax
import jax.numpy as jnp
from jax import lax
from jax.experimental import pallas as pl
from jax.experimental.pallas import tpu as pltpu
from jax.experimental.pallas import tpu_sc as plsc

N = 32768
D = 768
E = 8
LOSS_COEF = 0.01
GATE_EPS = 1e-10

BN = 4096
NW = 32
TPW = N // NW
NGROUPS = TPW // 16
BF = 4096


def _matmul_kernel(x_ref, wt_ref, bt_ref, geot_ref):
    geot_ref[:, :] = lax.dot_general(
        wt_ref[:, :], x_ref[:, :],
        dimension_numbers=(((1,), (1,)), ((), ())),
        preferred_element_type=jnp.float32,
    ) + bt_ref[:, :]


@jax.jit
def _matmul(x, WcatT, bcatT):
    return pl.pallas_call(
        _matmul_kernel,
        grid=(N // BN,),
        in_specs=[
            pl.BlockSpec((BN, D), lambda i: (i, 0)),
            pl.BlockSpec((2 * E, D), lambda i: (0, 0)),
            pl.BlockSpec((2 * E, 1), lambda i: (0, 0)),
        ],
        out_specs=pl.BlockSpec((2 * E, BN), lambda i: (0, i)),
        out_shape=jax.ShapeDtypeStruct((2 * E, N), jnp.float32),
    )(x, WcatT, bcatT)


def _routing_kernel(geot_hbm, out_hbm, gst_hbm, geot_v, pst_v, out_v):
    wid = lax.axis_index("s") * 2 + lax.axis_index("c")

    pltpu.sync_copy(geot_hbm.at[:, pl.ds(wid * TPW, TPW)], geot_v)

    zero16 = jnp.zeros((16,), jnp.float32)
    neg_inf = jnp.full((16,), -jnp.inf, jnp.float32)

    def group(t, carry):
        base = t * 16
        g = [geot_v[e, pl.ds(base, 16)] for e in range(E)]
        eo = [geot_v[E + e, pl.ds(base, 16)] for e in range(E)]

        m1 = g[0]
        for e in range(1, E):
            m1 = jnp.maximum(m1, g[e])
        a1 = jnp.full((16,), E, jnp.int32)
        for e in range(E - 1, -1, -1):
            a1 = jnp.where(g[e] == m1, jnp.full((16,), e, jnp.int32), a1)
        g2 = [jnp.where(a1 == e, neg_inf, g[e]) for e in range(E)]
        m2 = g2[0]
        for e in range(1, E):
            m2 = jnp.maximum(m2, g2[e])
        a2 = jnp.full((16,), E, jnp.int32)
        for e in range(E - 1, -1, -1):
            a2 = jnp.where(g2[e] == m2, jnp.full((16,), e, jnp.int32), a2)

        mx = jnp.maximum(m1, zero16)
        ex = []
        s = zero16
        for e in range(E):
            keep = (a1 == e) | (a2 == e)
            me = jnp.where(keep, g[e], zero16)
            x_e = jnp.exp(me - mx)
            ex.append(x_e)
            s = s + x_e
        r = 1.0 / s

        acc = zero16
        for e in range(E):
            p_e = ex[e] * r
            pst_v[e, pl.ds(base, 16)] = p_e
            acc = acc + p_e * eo[e]
        out_v[pl.ds(base, 16)] = acc
        return carry

    lax.fori_loop(0, NGROUPS, group, 0)

    pltpu.sync_copy(out_v, out_hbm.at[pl.ds(wid * TPW, TPW)])
    pltpu.sync_copy(pst_v, gst_hbm.at[:, pl.ds(wid * TPW, TPW)])


@jax.jit
def _routing(geot):
    f = functools.partial(
        pl.kernel,
        out_type=[
            jax.ShapeDtypeStruct((N,), jnp.float32),
            jax.ShapeDtypeStruct((E, N), jnp.float32),
        ],
        mesh=plsc.VectorSubcoreMesh(core_axis_name="c", subcore_axis_name="s"),
        compiler_params=pltpu.CompilerParams(skip_device_barrier=True),
        scratch_types=[
            pltpu.VMEM((2 * E, TPW), jnp.float32),
            pltpu.VMEM((E, TPW), jnp.float32),
            pltpu.VMEM((TPW,), jnp.float32),
        ],
    )(_routing_kernel)
    return f(geot)


def _finalize_kernel(gst_ref, gs_ref, loss_ref, imp_ref, load_ref):
    i = pl.program_id(0)
    nsteps = pl.num_programs(0)
    blk = gst_ref[:, :]
    gs_ref[:, :] = blk.T

    @pl.when(i == 0)
    def _():
        imp_ref[:, :] = jnp.zeros_like(imp_ref)
        load_ref[:, :] = jnp.zeros_like(load_ref)

    imp_ref[:, :] += jnp.sum(blk, axis=1, keepdims=True)
    load_ref[:, :] += jnp.sum((blk > 0).astype(jnp.float32), axis=1,
                              keepdims=True)

    @pl.when(i == nsteps - 1)
    def _():
        def cv2(v):
            mean = jnp.sum(v) / E
            var = jnp.sum((v - mean) ** 2) / (E - 1)
            return var / (mean * mean + GATE_EPS)

        loss = (cv2(imp_ref[:, :]) + cv2(load_ref[:, :])) * LOSS_COEF
        loss_ref[:, :] = jnp.full((1, 1), loss, dtype=jnp.float32)


@jax.jit
def _finalize(gst):
    gs, loss, _, _ = pl.pallas_call(
        _finalize_kernel,
        grid=(N // BF,),
        in_specs=[pl.BlockSpec((E, BF), lambda i: (0, i))],
        out_specs=[
            pl.BlockSpec((BF, E), lambda i: (i, 0)),
            pl.BlockSpec((1, 1), lambda i: (0, 0)),
            pl.BlockSpec((E, 1), lambda i: (0, 0)),
            pl.BlockSpec((E, 1), lambda i: (0, 0)),
        ],
        out_shape=[
            jax.ShapeDtypeStruct((N, E), jnp.float32),
            jax.ShapeDtypeStruct((1, 1), jnp.float32),
            jax.ShapeDtypeStruct((E, 1), jnp.float32),
            jax.ShapeDtypeStruct((E, 1), jnp.float32),
        ],
    )(gst)
    return gs, loss


def kernel(x, W_gate, b_gate, W_experts, b_experts):
    WcatT = jnp.concatenate([W_gate, W_experts], axis=1).T
    bcatT = jnp.concatenate([b_gate, b_experts]).reshape(2 * E, 1)
    geot = _matmul(x, WcatT, bcatT)
    out, gst = _routing(geot)
    gs, loss = _finalize(gst)
    return out, loss[0, 0], gs


# speedup vs baseline: 1.0006x; 1.0006x over previous
import functools

import jax
import jax.numpy as jnp
from jax import lax
from jax.experimental import pallas as pl
from jax.experimental.pallas import tpu as pltpu
from jax.experimental.pallas import tpu_sc as plsc

N = 32768
D = 768
E = 8
LOSS_COEF = 0.01
GATE_EPS = 1e-10

NW = 32
TPW = N // NW
NGROUPS = TPW // 16
BW = 4
BN = BW * TPW


def _matmul_kernel(x_ref, wt_ref, bt_ref, geo3_ref):
    res = lax.dot_general(
        wt_ref[:, :], x_ref[:, :],
        dimension_numbers=(((1,), (1,)), ((), ())),
        preferred_element_type=jnp.float32,
    ) + bt_ref[:, :]
    for w in range(BW):
        geo3_ref[w, :, :] = res[:, w * TPW:(w + 1) * TPW]


@jax.jit
def _matmul(x, WcatT, bcatT):
    return pl.pallas_call(
        _matmul_kernel,
        grid=(N // BN,),
        in_specs=[
            pl.BlockSpec((BN, D), lambda i: (i, 0)),
            pl.BlockSpec((2 * E, D), lambda i: (0, 0)),
            pl.BlockSpec((2 * E, 1), lambda i: (0, 0)),
        ],
        out_specs=pl.BlockSpec((BW, 2 * E, TPW), lambda i: (i, 0, 0)),
        out_shape=jax.ShapeDtypeStruct((NW, 2 * E, TPW), jnp.float32),
    )(x, WcatT, bcatT)


def _routing_kernel(geo3_hbm, out_hbm, gst3_hbm, geot_v, pst_v, out_v):
    wid = lax.axis_index("s") * 2 + lax.axis_index("c")

    pltpu.sync_copy(geo3_hbm.at[wid], geot_v)

    zero16 = jnp.zeros((16,), jnp.float32)
    neg_inf = jnp.full((16,), -jnp.inf, jnp.float32)

    def group(t, carry):
        base = t * 16
        g = [geot_v[e, pl.ds(base, 16)] for e in range(E)]
        eo = [geot_v[E + e, pl.ds(base, 16)] for e in range(E)]

        m1 = g[0]
        for e in range(1, E):
            m1 = jnp.maximum(m1, g[e])
        a1 = jnp.full((16,), E, jnp.int32)
        for e in range(E - 1, -1, -1):
            a1 = jnp.where(g[e] == m1, jnp.full((16,), e, jnp.int32), a1)
        g2 = [jnp.where(a1 == e, neg_inf, g[e]) for e in range(E)]
        m2 = g2[0]
        for e in range(1, E):
            m2 = jnp.maximum(m2, g2[e])
        a2 = jnp.full((16,), E, jnp.int32)
        for e in range(E - 1, -1, -1):
            a2 = jnp.where(g2[e] == m2, jnp.full((16,), e, jnp.int32), a2)

        mx = jnp.maximum(m1, zero16)
        ex = []
        s = zero16
        for e in range(E):
            keep = (a1 == e) | (a2 == e)
            me = jnp.where(keep, g[e], zero16)
            x_e = jnp.exp(me - mx)
            ex.append(x_e)
            s = s + x_e
        r = 1.0 / s

        acc = zero16
        for e in range(E):
            p_e = ex[e] * r
            pst_v[e, pl.ds(base, 16)] = p_e
            acc = acc + p_e * eo[e]
        out_v[pl.ds(base, 16)] = acc
        return carry

    lax.fori_loop(0, NGROUPS, group, 0)

    pltpu.sync_copy(out_v, out_hbm.at[pl.ds(wid * TPW, TPW)])
    pltpu.sync_copy(pst_v, gst3_hbm.at[wid])


@jax.jit
def _routing(geo3):
    f = functools.partial(
        pl.kernel,
        out_type=[
            jax.ShapeDtypeStruct((N,), jnp.float32),
            jax.ShapeDtypeStruct((NW, E, TPW), jnp.float32),
        ],
        mesh=plsc.VectorSubcoreMesh(core_axis_name="c", subcore_axis_name="s"),
        scratch_types=[
            pltpu.VMEM((2 * E, TPW), jnp.float32),
            pltpu.VMEM((E, TPW), jnp.float32),
            pltpu.VMEM((TPW,), jnp.float32),
        ],
    )(_routing_kernel)
    return f(geo3)


def _finalize_kernel(gst3_ref, gs_ref, loss_ref, imp_ref, load_ref):
    i = pl.program_id(0)
    nsteps = pl.num_programs(0)

    @pl.when(i == 0)
    def _():
        imp_ref[:, :] = jnp.zeros_like(imp_ref)
        load_ref[:, :] = jnp.zeros_like(load_ref)

    imp_acc = jnp.zeros((E, 1), jnp.float32)
    load_acc = jnp.zeros((E, 1), jnp.float32)
    for w in range(BW):
        blk = gst3_ref[w, :, :]
        gs_ref[pl.ds(w * TPW, TPW), :] = blk.T
        imp_acc += jnp.sum(blk, axis=1, keepdims=True)
        load_acc += jnp.sum((blk > 0).astype(jnp.float32), axis=1,
                            keepdims=True)
    imp_ref[:, :] += imp_acc
    load_ref[:, :] += load_acc

    @pl.when(i == nsteps - 1)
    def _():
        def cv2(v):
            mean = jnp.sum(v) / E
            var = jnp.sum((v - mean) ** 2) / (E - 1)
            return var / (mean * mean + GATE_EPS)

        loss = (cv2(imp_ref[:, :]) + cv2(load_ref[:, :])) * LOSS_COEF
        loss_ref[:, :] = jnp.full((1, 1), loss, dtype=jnp.float32)


@jax.jit
def _finalize(gst3):
    gs, loss, _, _ = pl.pallas_call(
        _finalize_kernel,
        grid=(NW // BW,),
        in_specs=[pl.BlockSpec((BW, E, TPW), lambda i: (i, 0, 0))],
        out_specs=[
            pl.BlockSpec((BW * TPW, E), lambda i: (i, 0)),
            pl.BlockSpec((1, 1), lambda i: (0, 0)),
            pl.BlockSpec((E, 1), lambda i: (0, 0)),
            pl.BlockSpec((E, 1), lambda i: (0, 0)),
        ],
        out_shape=[
            jax.ShapeDtypeStruct((N, E), jnp.float32),
            jax.ShapeDtypeStruct((1, 1), jnp.float32),
            jax.ShapeDtypeStruct((E, 1), jnp.float32),
            jax.ShapeDtypeStruct((E, 1), jnp.float32),
        ],
    )(gst3)
    return gs, loss


def kernel(x, W_gate, b_gate, W_experts, b_experts):
    WcatT = jnp.concatenate([W_gate, W_experts], axis=1).T
    bcatT = jnp.concatenate([b_gate, b_experts]).reshape(2 * E, 1)
    geo3 = _matmul(x, WcatT, bcatT)
    out, gst3 = _routing(geo3)
    gs, loss = _finalize(gst3)
    return out, loss[0, 0], gs


# speedup vs baseline: 1.0021x; 1.0015x over previous
import functools

import jax
import jax.numpy as jnp
from jax import lax
from jax.experimental import pallas as pl
from jax.experimental.pallas import tpu as pltpu
from jax.experimental.pallas import tpu_sc as plsc

N = 32768
D = 768
E = 8
LOSS_COEF = 0.01
GATE_EPS = 1e-10

BN = 4096
NW = 32
TPW = N // NW
NGROUPS = TPW // 16
BF = 4096


def _matmul_kernel(x_ref, wt_ref, bt_ref, geot_ref):
    geot_ref[:, :] = lax.dot_general(
        wt_ref[:, :], x_ref[:, :],
        dimension_numbers=(((1,), (1,)), ((), ())),
        preferred_element_type=jnp.float32,
    ) + bt_ref[:, :]


@jax.jit
def _matmul(x, WcatT, bcatT):
    return pl.pallas_call(
        _matmul_kernel,
        grid=(N // BN,),
        in_specs=[
            pl.BlockSpec((BN, D), lambda i: (i, 0)),
            pl.BlockSpec((2 * E, D), lambda i: (0, 0)),
            pl.BlockSpec((2 * E, 1), lambda i: (0, 0)),
        ],
        out_specs=pl.BlockSpec((2 * E, BN), lambda i: (0, i)),
        out_shape=jax.ShapeDtypeStruct((2 * E, N), jnp.float32),
    )(x, WcatT, bcatT)


def _routing_kernel(geot_hbm, out_hbm, gst_hbm, geot_v, pst_v, out_v):
    wid = lax.axis_index("s") * 2 + lax.axis_index("c")

    pltpu.sync_copy(geot_hbm.at[:, pl.ds(wid * TPW, TPW)], geot_v)

    zero16 = jnp.zeros((16,), jnp.float32)
    neg_inf = jnp.full((16,), -jnp.inf, jnp.float32)

    def group(t, carry):
        base = t * 16
        g = [geot_v[e, pl.ds(base, 16)] for e in range(E)]
        eo = [geot_v[E + e, pl.ds(base, 16)] for e in range(E)]

        m1 = g[0]
        for e in range(1, E):
            m1 = jnp.maximum(m1, g[e])
        a1 = jnp.full((16,), E, jnp.int32)
        for e in range(E - 1, -1, -1):
            a1 = jnp.where(g[e] == m1, jnp.full((16,), e, jnp.int32), a1)
        g2 = [jnp.where(a1 == e, neg_inf, g[e]) for e in range(E)]
        m2 = g2[0]
        for e in range(1, E):
            m2 = jnp.maximum(m2, g2[e])
        a2 = jnp.full((16,), E, jnp.int32)
        for e in range(E - 1, -1, -1):
            a2 = jnp.where(g2[e] == m2, jnp.full((16,), e, jnp.int32), a2)

        mx = jnp.maximum(m1, zero16)
        ex = []
        s = zero16
        for e in range(E):
            keep = (a1 == e) | (a2 == e)
            me = jnp.where(keep, g[e], zero16)
            x_e = jnp.exp(me - mx)
            ex.append(x_e)
            s = s + x_e
        r = 1.0 / s

        acc = zero16
        for e in range(E):
            p_e = ex[e] * r
            pst_v[e, pl.ds(base, 16)] = p_e
            acc = acc + p_e * eo[e]
        out_v[pl.ds(base, 16)] = acc
        return carry

    lax.fori_loop(0, NGROUPS, group, 0)

    pltpu.sync_copy(out_v, out_hbm.at[pl.ds(wid * TPW, TPW)])
    pltpu.sync_copy(pst_v, gst_hbm.at[:, pl.ds(wid * TPW, TPW)])


@jax.jit
def _routing(geot):
    f = functools.partial(
        pl.kernel,
        out_type=[
            jax.ShapeDtypeStruct((N,), jnp.float32),
            jax.ShapeDtypeStruct((E, N), jnp.float32),
        ],
        mesh=plsc.VectorSubcoreMesh(core_axis_name="c", subcore_axis_name="s"),
        scratch_types=[
            pltpu.VMEM((2 * E, TPW), jnp.float32),
            pltpu.VMEM((E, TPW), jnp.float32),
            pltpu.VMEM((TPW,), jnp.float32),
        ],
    )(_routing_kernel)
    return f(geot)


def _finalize_kernel(gst_ref, gs_ref, loss_ref, imp_ref, load_ref):
    i = pl.program_id(0)
    nsteps = pl.num_programs(0)
    blk = gst_ref[:, :]
    gs_ref[:, :] = blk.T

    @pl.when(i == 0)
    def _():
        imp_ref[:, :] = jnp.zeros_like(imp_ref)
        load_ref[:, :] = jnp.zeros_like(load_ref)

    imp_ref[:, :] += jnp.sum(blk, axis=1, keepdims=True)
    load_ref[:, :] += jnp.sum((blk > 0).astype(jnp.float32), axis=1,
                              keepdims=True)

    @pl.when(i == nsteps - 1)
    def _():
        def cv2(v):
            mean = jnp.sum(v) / E
            var = jnp.sum((v - mean) ** 2) / (E - 1)
            return var / (mean * mean + GATE_EPS)

        loss = (cv2(imp_ref[:, :]) + cv2(load_ref[:, :])) * LOSS_COEF
        loss_ref[:, :] = jnp.full((1, 1), loss, dtype=jnp.float32)


@jax.jit
def _finalize(gst):
    gs, loss, _, _ = pl.pallas_call(
        _finalize_kernel,
        grid=(N // BF,),
        in_specs=[pl.BlockSpec((E, BF), lambda i: (0, i))],
        out_specs=[
            pl.BlockSpec((BF, E), lambda i: (i, 0)),
            pl.BlockSpec((1, 1), lambda i: (0, 0)),
            pl.BlockSpec((E, 1), lambda i: (0, 0)),
            pl.BlockSpec((E, 1), lambda i: (0, 0)),
        ],
        out_shape=[
            jax.ShapeDtypeStruct((N, E), jnp.float32),
            jax.ShapeDtypeStruct((1, 1), jnp.float32),
            jax.ShapeDtypeStruct((E, 1), jnp.float32),
            jax.ShapeDtypeStruct((E, 1), jnp.float32),
        ],
    )(gst)
    return gs, loss


def kernel(x, W_gate, b_gate, W_experts, b_experts):
    WcatT = jnp.concatenate([W_gate, W_experts], axis=1).T
    bcatT = jnp.concatenate([b_gate, b_experts]).reshape(2 * E, 1)
    geot = _matmul(x, WcatT, bcatT)
    out, gst = _routing(geot)
    gs, loss = _finalize(gst)
    return out, loss[0, 0], gs


# speedup vs baseline: 1.0168x; 1.0147x over previous
import functools

import jax
import jax.numpy as jnp
from jax import lax
from jax.experimental import pallas as pl
from jax.experimental.pallas import tpu as pltpu
from jax.experimental.pallas import tpu_sc as plsc

N = 32768
D = 768
E = 8
LOSS_COEF = 0.01
GATE_EPS = 1e-10

BN = 4096
NW = 32
TPW = N // NW
NGROUPS = TPW // 16
BF = 8192


def _matmul_kernel(x_ref, wt_ref, bt_ref, geot_ref):
    geot_ref[:, :] = lax.dot_general(
        wt_ref[:, :], x_ref[:, :],
        dimension_numbers=(((1,), (1,)), ((), ())),
        preferred_element_type=jnp.float32,
    ) + bt_ref[:, :]


@jax.jit
def _matmul(x, WcatT, bcatT):
    return pl.pallas_call(
        _matmul_kernel,
        grid=(N // BN,),
        in_specs=[
            pl.BlockSpec((BN, D), lambda i: (i, 0)),
            pl.BlockSpec((2 * E, D), lambda i: (0, 0)),
            pl.BlockSpec((2 * E, 1), lambda i: (0, 0)),
        ],
        out_specs=pl.BlockSpec((2 * E, BN), lambda i: (0, i)),
        out_shape=jax.ShapeDtypeStruct((2 * E, N), jnp.float32),
    )(x, WcatT, bcatT)


def _routing_kernel(geot_hbm, out_hbm, gst_hbm, geot_v, pst_v, out_v):
    wid = lax.axis_index("s") * 2 + lax.axis_index("c")

    pltpu.sync_copy(geot_hbm.at[:, pl.ds(wid * TPW, TPW)], geot_v)

    zero16 = jnp.zeros((16,), jnp.float32)
    neg_inf = jnp.full((16,), -jnp.inf, jnp.float32)

    def group(t, carry):
        base = t * 16
        g = [geot_v[e, pl.ds(base, 16)] for e in range(E)]
        eo = [geot_v[E + e, pl.ds(base, 16)] for e in range(E)]

        m1 = g[0]
        for e in range(1, E):
            m1 = jnp.maximum(m1, g[e])
        a1 = jnp.full((16,), E, jnp.int32)
        for e in range(E - 1, -1, -1):
            a1 = jnp.where(g[e] == m1, jnp.full((16,), e, jnp.int32), a1)
        g2 = [jnp.where(a1 == e, neg_inf, g[e]) for e in range(E)]
        m2 = g2[0]
        for e in range(1, E):
            m2 = jnp.maximum(m2, g2[e])
        a2 = jnp.full((16,), E, jnp.int32)
        for e in range(E - 1, -1, -1):
            a2 = jnp.where(g2[e] == m2, jnp.full((16,), e, jnp.int32), a2)

        mx = jnp.maximum(m1, zero16)
        ex = []
        s = zero16
        for e in range(E):
            keep = (a1 == e) | (a2 == e)
            me = jnp.where(keep, g[e], zero16)
            x_e = jnp.exp(me - mx)
            ex.append(x_e)
            s = s + x_e
        r = 1.0 / s

        acc = zero16
        for e in range(E):
            p_e = ex[e] * r
            pst_v[e, pl.ds(base, 16)] = p_e
            acc = acc + p_e * eo[e]
        out_v[pl.ds(base, 16)] = acc
        return carry

    lax.fori_loop(0, NGROUPS, group, 0)

    pltpu.sync_copy(out_v, out_hbm.at[pl.ds(wid * TPW, TPW)])
    pltpu.sync_copy(pst_v, gst_hbm.at[:, pl.ds(wid * TPW, TPW)])


@jax.jit
def _routing(geot):
    f = functools.partial(
        pl.kernel,
        out_type=[
            jax.ShapeDtypeStruct((N,), jnp.float32),
            jax.ShapeDtypeStruct((E, N), jnp.float32),
        ],
        mesh=plsc.VectorSubcoreMesh(core_axis_name="c", subcore_axis_name="s"),
        scratch_types=[
            pltpu.VMEM((2 * E, TPW), jnp.float32),
            pltpu.VMEM((E, TPW), jnp.float32),
            pltpu.VMEM((TPW,), jnp.float32),
        ],
    )(_routing_kernel)
    return f(geot)


def _finalize_kernel(gst_ref, gs_ref, loss_ref, imp_ref, load_ref):
    i = pl.program_id(0)
    nsteps = pl.num_programs(0)
    blk = gst_ref[:, :]
    gs_ref[:, :] = blk.T

    @pl.when(i == 0)
    def _():
        imp_ref[:, :] = jnp.zeros_like(imp_ref)
        load_ref[:, :] = jnp.zeros_like(load_ref)

    imp_ref[:, :] += jnp.sum(blk, axis=1, keepdims=True)
    load_ref[:, :] += jnp.sum((blk > 0).astype(jnp.float32), axis=1,
                              keepdims=True)

    @pl.when(i == nsteps - 1)
    def _():
        def cv2(v):
            mean = jnp.sum(v) / E
            var = jnp.sum((v - mean) ** 2) / (E - 1)
            return var / (mean * mean + GATE_EPS)

        loss = (cv2(imp_ref[:, :]) + cv2(load_ref[:, :])) * LOSS_COEF
        loss_ref[:, :] = jnp.full((1, 1), loss, dtype=jnp.float32)


@jax.jit
def _finalize(gst):
    gs, loss, _, _ = pl.pallas_call(
        _finalize_kernel,
        grid=(N // BF,),
        in_specs=[pl.BlockSpec((E, BF), lambda i: (0, i))],
        out_specs=[
            pl.BlockSpec((BF, E), lambda i: (i, 0)),
            pl.BlockSpec((1, 1), lambda i: (0, 0)),
            pl.BlockSpec((E, 1), lambda i: (0, 0)),
            pl.BlockSpec((E, 1), lambda i: (0, 0)),
        ],
        out_shape=[
            jax.ShapeDtypeStruct((N, E), jnp.float32),
            jax.ShapeDtypeStruct((1, 1), jnp.float32),
            jax.ShapeDtypeStruct((E, 1), jnp.float32),
            jax.ShapeDtypeStruct((E, 1), jnp.float32),
        ],
    )(gst)
    return gs, loss


def kernel(x, W_gate, b_gate, W_experts, b_experts):
    WcatT = jnp.concatenate([W_gate, W_experts], axis=1).T
    bcatT = jnp.concatenate([b_gate, b_experts]).reshape(2 * E, 1)
    geot = _matmul(x, WcatT, bcatT)
    out, gst = _routing(geot)
    gs, loss = _finalize(gst)
    return out, loss[0, 0], gs
